# baseline (speedup 1.0000x reference)
import functools

import jax
import jax.numpy as jnp
from jax import lax
from jax.experimental import pallas as pl
from jax.experimental.pallas import tpu as pltpu
from jax.experimental.pallas import tpu_sc as plsc

_NUM_WORKERS = 32
_ROWS = 16


@functools.lru_cache(maxsize=None)
def _make_gather(V, D, Bm, F):
    B = Bm * F
    R = _ROWS
    C = R * F
    rows_per_w = Bm // _NUM_WORKERS
    b_per_w = rows_per_w * F
    n_chunks = rows_per_w // R
    n_pairs = n_chunks // 2
    mesh = plsc.VectorSubcoreMesh(core_axis_name="c", subcore_axis_name="s")

    @functools.partial(
        pl.kernel,
        mesh=mesh,
        compiler_params=pltpu.CompilerParams(use_tc_tiling_on_sc=False),
        out_type=jax.ShapeDtypeStruct((Bm, F, D), jnp.float32),
        scratch_types=[
            pltpu.VMEM((b_per_w,), jnp.int32),
            pltpu.VMEM((C, D), jnp.float32),
            pltpu.VMEM((C, D), jnp.float32),
            pltpu.SemaphoreType.DMA,
            pltpu.SemaphoreType.DMA,
        ],
    )
    def gather_kernel(idx_hbm, table_hbm, out_hbm, idx_v, r0, r1, gsem, wsem):
        wid = lax.axis_index("s") * 2 + lax.axis_index("c")
        base = wid * b_per_w
        row_base = wid * rows_per_w
        pltpu.sync_copy(idx_hbm.at[pl.ds(base, b_per_w)], idx_v)

        def fire_gather(g, buf):
            pltpu.async_copy(table_hbm.at[idx_v.at[pl.ds(g * C, C)]], buf, gsem)

        def wait_gather(buf):
            pltpu.make_async_copy(table_hbm.at[pl.ds(0, C)], buf, gsem).wait()

        def fire_write(g, buf):
            for j in range(R):
                pltpu.async_copy(
                    buf.at[pl.ds(j * F, F)], out_hbm.at[row_base + g * R + j], wsem)

        def wait_write(buf):
            pltpu.make_async_copy(table_hbm.at[pl.ds(0, C)], buf, wsem).wait()

        fire_gather(0, r0)
        fire_gather(1, r1)
        wait_gather(r0)
        fire_write(0, r0)
        wait_gather(r1)
        fire_write(1, r1)

        def body(i, carry):
            g0 = 2 * i
            wait_write(r0)
            fire_gather(g0, r0)
            wait_gather(r0)
            fire_write(g0, r0)
            wait_write(r1)
            fire_gather(g0 + 1, r1)
            wait_gather(r1)
            fire_write(g0 + 1, r1)
            return carry

        lax.fori_loop(1, n_pairs, body, 0)
        wait_write(r0)
        wait_write(r1)

    return gather_kernel


def kernel(input, W):
    Bm, F = input.shape
    V, D = W.shape
    idx_flat = input.reshape(Bm * F)
    return _make_gather(V, D, Bm, F)(idx_flat, W)


# speedup vs baseline: 4.6239x; 4.6239x over previous
import functools

import jax
import jax.numpy as jnp
from jax import lax
from jax.experimental import pallas as pl
from jax.experimental.pallas import tpu as pltpu
from jax.experimental.pallas import tpu_sc as plsc

_NUM_WORKERS = 32
_L = 16


@functools.lru_cache(maxsize=None)
def _make_gather(V, D, Bm, F):
    DR = D // 8
    BC = Bm // 128
    bc_per_w = BC // _NUM_WORKERS
    bpw = bc_per_w * 128
    Vp = 1024
    mesh = plsc.VectorSubcoreMesh(core_axis_name="c", subcore_axis_name="s")

    @functools.partial(
        pl.kernel,
        mesh=mesh,
        compiler_params=pltpu.CompilerParams(
            use_tc_tiling_on_sc=False, needs_layout_passes=False),
        out_type=jax.ShapeDtypeStruct((F, DR, BC, 8, 128), jnp.float32),
        scratch_types=[
            pltpu.VMEM((D * Vp,), jnp.float32),
            pltpu.VMEM((bpw,), jnp.int32),
            pltpu.VMEM((bc_per_w, 8, 128), jnp.float32),
            pltpu.VMEM((bc_per_w, 8, 128), jnp.float32),
            pltpu.SemaphoreType.DMA,
        ],
    )
    def gather_kernel(idxT_hbm, wt_hbm, out_hbm, wt_v, idx_v, o0, o1, wsem):
        wid = lax.axis_index("s") * 2 + lax.axis_index("c")
        pltpu.sync_copy(wt_hbm, wt_v)
        n_blk = bpw // _L

        def wait_write(buf):
            pltpu.make_async_copy(buf, out_hbm.at[0, 0, pl.ds(0, bc_per_w)],
                                  wsem).wait()

        def body(f, carry):
            pltpu.sync_copy(idxT_hbm.at[pl.ds(f * Bm + wid * bpw, bpw)], idx_v)
            for dr in range(DR):
                buf = o0 if dr % 2 == 0 else o1
                if dr >= 2:
                    wait_write(buf)
                else:
                    @pl.when(f > 0)
                    def _():
                        wait_write(buf)

                @plsc.parallel_loop(0, n_blk, unroll=4)
                def blk(k):
                    bc = lax.shift_right_logical(k, 3)
                    g16 = lax.mul(lax.bitwise_and(k, 7), _L)
                    iv = idx_v[pl.ds(k * _L, _L)]
                    for ds in range(8):
                        iv2 = iv + jnp.int32((dr * 8 + ds) * Vp)
                        vals = plsc.load_gather(wt_v, [iv2])
                        buf[bc, ds, pl.ds(g16, _L)] = vals

                pltpu.async_copy(
                    buf, out_hbm.at[f, dr, pl.ds(wid * bc_per_w, bc_per_w)],
                    wsem)
            return carry

        lax.fori_loop(0, F, body, 0)
        wait_write(o0)
        wait_write(o1)

    return gather_kernel


def kernel(input, W):
    Bm, F = input.shape
    V, D = W.shape
    idxT_flat = input.T.reshape(Bm * F)
    wt_pad = jnp.pad(W.T, ((0, 0), (0, 1024 - V))).reshape(D * 1024)
    o5 = _make_gather(V, D, Bm, F)(idxT_flat, wt_pad)
    return o5.transpose(2, 4, 0, 1, 3).reshape(Bm, F, D)


# speedup vs baseline: 4.7152x; 1.0197x over previous
import functools

import jax
import jax.numpy as jnp
from jax import lax
from jax.experimental import pallas as pl
from jax.experimental.pallas import tpu as pltpu
from jax.experimental.pallas import tpu_sc as plsc

_NUM_WORKERS = 32
_L = 16


@functools.lru_cache(maxsize=None)
def _make_gather(V, D, Bm, F):
    DR = D // 8
    BC = Bm // 128
    bc_per_w = BC // _NUM_WORKERS
    bpw = bc_per_w * 128
    Vp = 1024
    mesh = plsc.VectorSubcoreMesh(core_axis_name="c", subcore_axis_name="s")

    @functools.partial(
        pl.kernel,
        mesh=mesh,
        compiler_params=pltpu.CompilerParams(
            use_tc_tiling_on_sc=False, needs_layout_passes=False),
        out_type=jax.ShapeDtypeStruct((F, DR, BC, 8, 128), jnp.float32),
        scratch_types=[
            pltpu.VMEM((D * Vp,), jnp.float32),
            pltpu.VMEM((bpw,), jnp.int32),
            pltpu.VMEM((bc_per_w, 8, 128), jnp.float32),
            pltpu.VMEM((bc_per_w, 8, 128), jnp.float32),
            pltpu.SemaphoreType.DMA,
        ],
    )
    def gather_kernel(idxT_hbm, wt_hbm, out_hbm, wt_v, idx_v, o0, o1, wsem):
        wid = lax.axis_index("s") * 2 + lax.axis_index("c")
        pltpu.sync_copy(wt_hbm, wt_v)
        n_blk = bpw // _L

        def wait_write(buf):
            pltpu.make_async_copy(buf, out_hbm.at[0, 0, pl.ds(0, bc_per_w)],
                                  wsem).wait()

        def body(f, carry):
            pltpu.sync_copy(idxT_hbm.at[pl.ds(f * Bm + wid * bpw, bpw)], idx_v)
            for dr in range(DR):
                buf = o0 if dr % 2 == 0 else o1
                if dr >= 2:
                    wait_write(buf)
                else:
                    @pl.when(f > 0)
                    def _():
                        wait_write(buf)

                @plsc.parallel_loop(0, n_blk, unroll=8)
                def blk(k):
                    bc = lax.shift_right_logical(k, 3)
                    g16 = lax.mul(lax.bitwise_and(k, 7), _L)
                    iv = idx_v[pl.ds(k * _L, _L)]
                    for ds in range(8):
                        iv2 = iv + jnp.int32((dr * 8 + ds) * Vp)
                        vals = plsc.load_gather(wt_v, [iv2])
                        buf[bc, ds, pl.ds(g16, _L)] = vals

                pltpu.async_copy(
                    buf, out_hbm.at[f, dr, pl.ds(wid * bc_per_w, bc_per_w)],
                    wsem)
            return carry

        lax.fori_loop(0, F, body, 0)
        wait_write(o0)
        wait_write(o1)

    return gather_kernel


def kernel(input, W):
    Bm, F = input.shape
    V, D = W.shape
    idxT_flat = input.T.reshape(Bm * F)
    wt_pad = jnp.pad(W.T, ((0, 0), (0, 1024 - V))).reshape(D * 1024)
    o5 = _make_gather(V, D, Bm, F)(idxT_flat, wt_pad)
    return o5.transpose(2, 4, 0, 1, 3).reshape(Bm, F, D)
